# baseline (speedup 1.0000x reference)
import functools

import jax
import jax.numpy as jnp
from jax import lax
from jax.experimental import pallas as pl
from jax.experimental.pallas import tpu as pltpu
from jax.experimental.pallas import tpu_sc as plsc

N = 10000
NNZ = 320000
D = 128

NC, NS, LANES = 2, 16, 16
CH = 128
EPC = NNZ // NC
NCHUNKS = EPC // CH
ITERS = (NCHUNKS + NS - 1) // NS
CPC = NS * ITERS
PAD = CPC - NCHUNKS
NA = N + 8
BAT = 8
KD = D // LANES
WCH = 80
NWCH = N // WCH
WITER = (NWCH + NS - 1) // NS


def _zero_buf2d(buf, n):
    zval = jnp.zeros((LANES,), jnp.float32)

    def zrow(i, _):
        buf[i // KD, pl.ds((i % KD) * LANES, LANES)] = zval
        return 0

    lax.fori_loop(0, n * KD, zrow, 0)


def _make_sc_agg(with_deg):
    mesh = plsc.VectorSubcoreMesh(core_axis_name="c", subcore_axis_name="s")

    out_type = [jax.ShapeDtypeStruct((2 * N, D), jnp.float32)]
    scratch = [
        pltpu.VMEM((CH, D), jnp.float32),
        pltpu.VMEM_SHARED((NA, D), jnp.float32),
        pltpu.SemaphoreType.DMA,
    ]
    if with_deg:
        out_type += [jax.ShapeDtypeStruct((2 * N,), jnp.float32),
                     jax.ShapeDtypeStruct((2 * N,), jnp.float32)]
        scratch += [
            pltpu.VMEM((CH,), jnp.int32),
            pltpu.VMEM((CH,), jnp.int32),
            pltpu.VMEM((CH,), jnp.float32),
            pltpu.VMEM((CH,), jnp.float32),
            pltpu.VMEM_SHARED((N,), jnp.float32),
            pltpu.VMEM_SHARED((NA,), jnp.float32),
            pltpu.SemaphoreType.DMA,
        ]
    else:
        scratch += [
            pltpu.VMEM((CH,), jnp.int32),
            pltpu.VMEM((CH,), jnp.int32),
        ]

    def body(refs):
        if with_deg:
            (table, gidx, sidx, ew, out, outd, outde,
             rows, acc, sem, gbuf, sbuf, vals, ones,
             accd, accde, sem2) = refs
        else:
            table, gidx, sidx, out, rows, acc, sem, gbuf, sbuf = refs
        c = lax.axis_index("c")
        s = lax.axis_index("s")

        _zero_buf2d(rows, CH)
        if with_deg:
            zv = jnp.zeros((LANES,), jnp.float32)
            ov = jnp.ones((LANES,), jnp.float32)
            for k in range(CH // LANES):
                vals[pl.ds(k * LANES, LANES)] = zv
                ones[pl.ds(k * LANES, LANES)] = ov
        for t in range(WITER):
            wid = t * NS + s

            @pl.when(wid < NWCH)
            def _():
                pltpu.sync_copy(rows.at[pl.ds(0, WCH)],
                                acc.at[pl.ds(wid * WCH, WCH)])
                if with_deg:
                    pltpu.sync_copy(vals.at[pl.ds(0, WCH)],
                                    accd.at[pl.ds(wid * WCH, WCH)])
                    pltpu.sync_copy(vals.at[pl.ds(0, WCH)],
                                    accde.at[pl.ds(wid * WCH, WCH)])

        plsc.subcore_barrier()

        if with_deg:
            def chunk(i, _):
                base = c * CPC * CH + (i * NS + s) * CH
                pltpu.sync_copy(gidx.at[pl.ds(base, CH)], gbuf)
                pltpu.sync_copy(sidx.at[pl.ds(base, CH)], sbuf)
                g = pltpu.async_copy(table.at[gbuf], rows, sem)
                pltpu.async_copy(ew.at[sbuf], vals, sem2).wait()
                pltpu.sync_copy(vals, accd.at[gbuf], add=True)
                pltpu.sync_copy(ones, accde.at[sbuf], add=True)
                g.wait()
                pltpu.sync_copy(rows, acc.at[sbuf], add=True)
                return 0

            lax.fori_loop(0, ITERS, chunk, 0)
        else:
            def chunk(i, _):
                base = c * CPC * CH + (i * NS + s) * CH
                pltpu.sync_copy(gidx.at[pl.ds(base, CH)], gbuf)
                pltpu.sync_copy(sidx.at[pl.ds(base, CH)], sbuf)
                pltpu.async_copy(table.at[gbuf], rows, sem).wait()
                pltpu.sync_copy(rows, acc.at[sbuf], add=True)
                return 0

            lax.fori_loop(0, ITERS, chunk, 0)
        plsc.subcore_barrier()

        for t in range(WITER):
            wid = t * NS + s

            @pl.when(wid < NWCH)
            def _():
                r0 = wid * WCH
                pltpu.sync_copy(acc.at[pl.ds(r0, WCH)], rows.at[pl.ds(0, WCH)])
                pltpu.sync_copy(rows.at[pl.ds(0, WCH)],
                                out.at[pl.ds(c * N + r0, WCH)])
                if with_deg:
                    pltpu.sync_copy(accd.at[pl.ds(r0, WCH)],
                                    vals.at[pl.ds(0, WCH)])
                    pltpu.sync_copy(vals.at[pl.ds(0, WCH)],
                                    outd.at[pl.ds(c * N + r0, WCH)])
                    pltpu.sync_copy(accde.at[pl.ds(r0, WCH)],
                                    vals.at[pl.ds(0, WCH)])
                    pltpu.sync_copy(vals.at[pl.ds(0, WCH)],
                                    outde.at[pl.ds(c * N + r0, WCH)])

    def wrap(*args):
        return pl.kernel(
            lambda *refs: body(refs),
            out_type=tuple(out_type) if with_deg else out_type[0],
            mesh=mesh,
            scratch_types=scratch,
        )(*args)

    return wrap


_sc_agg_deg = _make_sc_agg(True)
_sc_agg = _make_sc_agg(False)


BN = 1000
RB = N // BN


def _inv(v):
    return jnp.where(v > 0, 1.0 / jnp.where(v > 0, v, 1.0), 0.0)


def _scale_body(a0_ref, a1_ref, d0_ref, d1_ref, o_ref):
    deg = d0_ref[0, 0, 0, :] + d1_ref[0, 0, 0, :]
    o_ref[...] = (a0_ref[...] + a1_ref[...]) * _inv(deg)[:, None]


def _scale(P, degp):
    return pl.pallas_call(
        _scale_body,
        grid=(RB,),
        in_specs=[pl.BlockSpec((BN, D), lambda r: (r, 0)),
                  pl.BlockSpec((BN, D), lambda r: (RB + r, 0)),
                  pl.BlockSpec((1, 1, 1, BN), lambda r: (0, r, 0, 0)),
                  pl.BlockSpec((1, 1, 1, BN), lambda r: (1, r, 0, 0))],
        out_specs=pl.BlockSpec((BN, D), lambda r: (r, 0)),
        out_shape=jax.ShapeDtypeStruct((N, D), jnp.float32),
    )(P, P, degp, degp)


def _mmrelu_body(a0_ref, a1_ref, d0_ref, d1_ref, w_ref, b_ref, o_ref):
    d = d0_ref[0, 0, 0, :] + d1_ref[0, 0, 0, :]
    v = (a0_ref[...] + a1_ref[...]) * _inv(d)[:, None]
    o_ref[...] = jnp.maximum(
        jnp.dot(v, w_ref[...], preferred_element_type=jnp.float32)
        + b_ref[0, :][None, :], 0.0)


def _mmrelu(P, dp, W, b, DO):
    cb = DO // 128
    return pl.pallas_call(
        _mmrelu_body,
        grid=(cb, RB),
        in_specs=[pl.BlockSpec((BN, D), lambda c, r: (r, 0)),
                  pl.BlockSpec((BN, D), lambda c, r: (RB + r, 0)),
                  pl.BlockSpec((1, 1, 1, BN), lambda c, r: (0, r, 0, 0)),
                  pl.BlockSpec((1, 1, 1, BN), lambda c, r: (1, r, 0, 0)),
                  pl.BlockSpec((D, 128), lambda c, r: (0, c)),
                  pl.BlockSpec((1, 128), lambda c, r: (0, c))],
        out_specs=pl.BlockSpec((BN, 128), lambda c, r: (r, c)),
        out_shape=jax.ShapeDtypeStruct((N, DO), jnp.float32),
    )(P, P, dp, dp, W, b.reshape(1, DO))


def _pad1d(a, padval):
    h = a.reshape(2, EPC)
    p = jnp.full((2, PAD * CH), padval, jnp.int32)
    return jnp.concatenate([h, p], axis=1).reshape(-1)


def kernel(x, edge_index, edge_weight, batch, W1, b1, W2, b2):
    row = edge_index[0].astype(jnp.int32)
    col = edge_index[1].astype(jnp.int32)
    ew = edge_weight.astype(jnp.float32)

    row_g = _pad1d(row, 0)
    row_s = _pad1d(row, N)
    col_g = _pad1d(col, 0)
    col_s = _pad1d(col, N)
    ew_p = jnp.concatenate([ew, jnp.zeros((8,), jnp.float32)])

    P1, dpart, depart = _sc_agg_deg(x, row_g, col_s, ew_p)
    dp = dpart.reshape(2, RB, 1, BN)
    dep = depart.reshape(2, RB, 1, BN)
    T2 = _scale(P1, dep)
    P2 = _sc_agg(T2, col_g, row_s)
    h1 = _mmrelu(P2, dp, W1, b1, D)

    P3 = _sc_agg(h1, row_g, col_s)
    T4 = _scale(P3, dep)
    P4 = _sc_agg(T4, col_g, row_s)
    return _mmrelu(P4, dp, W2, b2, 2 * D)


# speedup vs baseline: 1.4581x; 1.4581x over previous
import functools

import jax
import jax.numpy as jnp
from jax import lax
from jax.experimental import pallas as pl
from jax.experimental.pallas import tpu as pltpu
from jax.experimental.pallas import tpu_sc as plsc

N = 10000
NNZ = 320000
D = 128

NC, NS, LANES = 2, 16, 16
CH = 128
EPC = NNZ // NC
NCHUNKS = EPC // CH
ITERS = (NCHUNKS + NS - 1) // NS
KD = D // LANES
WCH = 80
NWCH = N // WCH
WITER = (NWCH + NS - 1) // NS


def _zero_buf2d(buf, n):
    zval = jnp.zeros((LANES,), jnp.float32)

    def zrow(i, _):
        buf[i // KD, pl.ds((i % KD) * LANES, LANES)] = zval
        return 0

    lax.fori_loop(0, n * KD, zrow, 0)


def _make_sc_agg(with_deg):
    mesh = plsc.VectorSubcoreMesh(core_axis_name="c", subcore_axis_name="s")

    out_type = [jax.ShapeDtypeStruct((2 * N, D), jnp.float32)]
    scratch = [
        pltpu.VMEM((CH, D), jnp.float32),
        pltpu.VMEM((CH,), jnp.int32),
        pltpu.VMEM((CH,), jnp.int32),
        pltpu.VMEM_SHARED((N, D), jnp.float32),
        pltpu.SemaphoreType.DMA,
    ]
    if with_deg:
        out_type += [jax.ShapeDtypeStruct((2 * N,), jnp.float32),
                     jax.ShapeDtypeStruct((2 * N,), jnp.float32)]
        scratch += [
            pltpu.VMEM((CH,), jnp.float32),
            pltpu.VMEM((CH,), jnp.float32),
            pltpu.VMEM_SHARED((N,), jnp.float32),
            pltpu.VMEM_SHARED((N,), jnp.float32),
            pltpu.SemaphoreType.DMA,
        ]

    def body(refs):
        if with_deg:
            (table, gidx, sidx, ew, out, outd, outde,
             rows, gbuf, sbuf, acc, sem, vals, ones, accd, accde, sem2) = refs
        else:
            table, gidx, sidx, out, rows, gbuf, sbuf, acc, sem = refs
        c = lax.axis_index("c")
        s = lax.axis_index("s")

        _zero_buf2d(rows, CH)
        if with_deg:
            zv = jnp.zeros((LANES,), jnp.float32)
            ov = jnp.ones((LANES,), jnp.float32)
            for k in range(CH // LANES):
                vals[pl.ds(k * LANES, LANES)] = zv
                ones[pl.ds(k * LANES, LANES)] = ov
        for t in range(WITER):
            wid = t * NS + s

            @pl.when(wid < NWCH)
            def _():
                pltpu.sync_copy(rows.at[pl.ds(0, WCH)],
                                acc.at[pl.ds(wid * WCH, WCH)])
                if with_deg:
                    pltpu.sync_copy(vals.at[pl.ds(0, WCH)],
                                    accd.at[pl.ds(wid * WCH, WCH)])
                    pltpu.sync_copy(vals.at[pl.ds(0, WCH)],
                                    accde.at[pl.ds(wid * WCH, WCH)])

        plsc.subcore_barrier()

        def chunk(i, _):
            cid = i * NS + s

            @pl.when(cid < NCHUNKS)
            def _():
                base = c * EPC + cid * CH
                pltpu.sync_copy(gidx.at[pl.ds(base, CH)], gbuf)
                pltpu.sync_copy(sidx.at[pl.ds(base, CH)], sbuf)
                g = pltpu.async_copy(table.at[gbuf], rows, sem)
                if with_deg:
                    pltpu.async_copy(ew.at[sbuf], vals, sem2).wait()
                    pltpu.sync_copy(vals, accd.at[gbuf], add=True)
                    pltpu.sync_copy(ones, accde.at[sbuf], add=True)
                g.wait()
                pltpu.sync_copy(rows, acc.at[sbuf], add=True)

            return 0

        lax.fori_loop(0, ITERS, chunk, 0)
        plsc.subcore_barrier()

        for t in range(WITER):
            wid = t * NS + s

            @pl.when(wid < NWCH)
            def _():
                r0 = wid * WCH
                pltpu.sync_copy(acc.at[pl.ds(r0, WCH)], rows.at[pl.ds(0, WCH)])
                pltpu.sync_copy(rows.at[pl.ds(0, WCH)],
                                out.at[pl.ds(c * N + r0, WCH)])
                if with_deg:
                    pltpu.sync_copy(accd.at[pl.ds(r0, WCH)],
                                    vals.at[pl.ds(0, WCH)])
                    pltpu.sync_copy(vals.at[pl.ds(0, WCH)],
                                    outd.at[pl.ds(c * N + r0, WCH)])
                    pltpu.sync_copy(accde.at[pl.ds(r0, WCH)],
                                    vals.at[pl.ds(0, WCH)])
                    pltpu.sync_copy(vals.at[pl.ds(0, WCH)],
                                    outde.at[pl.ds(c * N + r0, WCH)])

    def wrap(*args):
        return pl.kernel(
            lambda *refs: body(refs),
            out_type=tuple(out_type) if with_deg else out_type[0],
            mesh=mesh,
            scratch_types=scratch,
        )(*args)

    return wrap


_sc_agg_deg = _make_sc_agg(True)
_sc_agg = _make_sc_agg(False)


BN = 1000
RB = N // BN


def _inv(v):
    return jnp.where(v > 0, 1.0 / jnp.where(v > 0, v, 1.0), 0.0)


def _scale_body(a0_ref, a1_ref, d0_ref, d1_ref, o_ref):
    deg = d0_ref[0, 0, 0, :] + d1_ref[0, 0, 0, :]
    o_ref[...] = (a0_ref[...] + a1_ref[...]) * _inv(deg)[:, None]


def _scale(P, degp):
    return pl.pallas_call(
        _scale_body,
        grid=(RB,),
        in_specs=[pl.BlockSpec((BN, D), lambda r: (r, 0)),
                  pl.BlockSpec((BN, D), lambda r: (RB + r, 0)),
                  pl.BlockSpec((1, 1, 1, BN), lambda r: (0, r, 0, 0)),
                  pl.BlockSpec((1, 1, 1, BN), lambda r: (1, r, 0, 0))],
        out_specs=pl.BlockSpec((BN, D), lambda r: (r, 0)),
        out_shape=jax.ShapeDtypeStruct((N, D), jnp.float32),
    )(P, P, degp, degp)


def _mmrelu_body(a0_ref, a1_ref, d0_ref, d1_ref, w_ref, b_ref, o_ref):
    d = d0_ref[0, 0, 0, :] + d1_ref[0, 0, 0, :]
    v = (a0_ref[...] + a1_ref[...]) * _inv(d)[:, None]
    o_ref[...] = jnp.maximum(
        jnp.dot(v, w_ref[...], preferred_element_type=jnp.float32)
        + b_ref[0, :][None, :], 0.0)


def _mmrelu(P, dp, W, b, DO):
    cb = DO // 128
    return pl.pallas_call(
        _mmrelu_body,
        grid=(cb, RB),
        in_specs=[pl.BlockSpec((BN, D), lambda c, r: (r, 0)),
                  pl.BlockSpec((BN, D), lambda c, r: (RB + r, 0)),
                  pl.BlockSpec((1, 1, 1, BN), lambda c, r: (0, r, 0, 0)),
                  pl.BlockSpec((1, 1, 1, BN), lambda c, r: (1, r, 0, 0)),
                  pl.BlockSpec((D, 128), lambda c, r: (0, c)),
                  pl.BlockSpec((1, 128), lambda c, r: (0, c))],
        out_specs=pl.BlockSpec((BN, 128), lambda c, r: (r, c)),
        out_shape=jax.ShapeDtypeStruct((N, DO), jnp.float32),
    )(P, P, dp, dp, W, b.reshape(1, DO))


def kernel(x, edge_index, edge_weight, batch, W1, b1, W2, b2):
    row = edge_index[0].astype(jnp.int32)
    col = edge_index[1].astype(jnp.int32)
    ew = edge_weight.astype(jnp.float32)

    P1, dpart, depart = _sc_agg_deg(x, row, col, ew)
    dp = dpart.reshape(2, RB, 1, BN)
    dep = depart.reshape(2, RB, 1, BN)
    T2 = _scale(P1, dep)
    P2 = _sc_agg(T2, col, row)
    h1 = _mmrelu(P2, dp, W1, b1, D)

    P3 = _sc_agg(h1, row, col)
    T4 = _scale(P3, dep)
    P4 = _sc_agg(T4, col, row)
    return _mmrelu(P4, dp, W2, b2, 2 * D)


# speedup vs baseline: 1.6570x; 1.1364x over previous
import functools

import jax
import jax.numpy as jnp
from jax import lax
from jax.experimental import pallas as pl
from jax.experimental.pallas import tpu as pltpu
from jax.experimental.pallas import tpu_sc as plsc

N = 10000
NNZ = 320000
D = 128

NC, NS, LANES = 2, 16, 16
CH = 128
EPC = NNZ // NC
NCHUNKS = EPC // CH
ITERS = (NCHUNKS + NS - 1) // NS
KD = D // LANES
WCH = 80
NWCH = N // WCH
WITER = (NWCH + NS - 1) // NS


def _zero_buf2d(buf, n):
    zval = jnp.zeros((LANES,), jnp.float32)

    def zrow(i, _):
        buf[i // KD, pl.ds((i % KD) * LANES, LANES)] = zval
        return 0

    lax.fori_loop(0, n * KD, zrow, 0)


def _make_sc_agg(with_deg):
    mesh = plsc.VectorSubcoreMesh(core_axis_name="c", subcore_axis_name="s")

    out_type = [jax.ShapeDtypeStruct((2 * N, D), jnp.float32)]
    scratch = [
        pltpu.VMEM((CH, D), jnp.float32),
        pltpu.VMEM((CH,), jnp.int32),
        pltpu.VMEM((CH,), jnp.int32),
        pltpu.VMEM_SHARED((N, D), jnp.float32),
        pltpu.SemaphoreType.DMA,
    ]
    if with_deg:
        out_type += [jax.ShapeDtypeStruct((2 * N,), jnp.float32),
                     jax.ShapeDtypeStruct((2 * N,), jnp.float32)]
        scratch += [
            pltpu.VMEM((CH,), jnp.float32),
            pltpu.VMEM((CH,), jnp.float32),
            pltpu.VMEM_SHARED((N,), jnp.float32),
            pltpu.VMEM_SHARED((N,), jnp.float32),
            pltpu.SemaphoreType.DMA,
        ]

    def body(refs):
        if with_deg:
            (table, gidx, sidx, ew, out, outd, outde,
             rows, gbuf, sbuf, acc, sem, vals, ones, accd, accde, sem2) = refs
        else:
            table, gidx, sidx, out, rows, gbuf, sbuf, acc, sem = refs
        c = lax.axis_index("c")
        s = lax.axis_index("s")

        _zero_buf2d(rows, CH)
        if with_deg:
            zv = jnp.zeros((LANES,), jnp.float32)
            ov = jnp.ones((LANES,), jnp.float32)
            for k in range(CH // LANES):
                vals[pl.ds(k * LANES, LANES)] = zv
                ones[pl.ds(k * LANES, LANES)] = ov
        for t in range(WITER):
            wid = t * NS + s

            @pl.when(wid < NWCH)
            def _():
                pltpu.sync_copy(rows.at[pl.ds(0, WCH)],
                                acc.at[pl.ds(wid * WCH, WCH)])
                if with_deg:
                    pltpu.sync_copy(vals.at[pl.ds(0, WCH)],
                                    accd.at[pl.ds(wid * WCH, WCH)])
                    pltpu.sync_copy(vals.at[pl.ds(0, WCH)],
                                    accde.at[pl.ds(wid * WCH, WCH)])

        plsc.subcore_barrier()

        def chunk(i, _):
            cid = i * NS + s

            @pl.when(cid < NCHUNKS)
            def _():
                base = c * EPC + cid * CH
                pltpu.sync_copy(gidx.at[pl.ds(base, CH)], gbuf)
                g = pltpu.async_copy(table.at[gbuf], rows, sem)
                pltpu.sync_copy(sidx.at[pl.ds(base, CH)], sbuf)
                if with_deg:
                    pltpu.async_copy(ew.at[sbuf], vals, sem2).wait()
                    pltpu.sync_copy(vals, accd.at[gbuf], add=True)
                    pltpu.sync_copy(ones, accde.at[sbuf], add=True)
                g.wait()
                pltpu.sync_copy(rows, acc.at[sbuf], add=True)

            return 0

        lax.fori_loop(0, ITERS, chunk, 0)
        plsc.subcore_barrier()

        for t in range(WITER):
            wid = t * NS + s

            @pl.when(wid < NWCH)
            def _():
                r0 = wid * WCH
                pltpu.sync_copy(acc.at[pl.ds(r0, WCH)], rows.at[pl.ds(0, WCH)])
                pltpu.sync_copy(rows.at[pl.ds(0, WCH)],
                                out.at[pl.ds(c * N + r0, WCH)])
                if with_deg:
                    pltpu.sync_copy(accd.at[pl.ds(r0, WCH)],
                                    vals.at[pl.ds(0, WCH)])
                    pltpu.sync_copy(vals.at[pl.ds(0, WCH)],
                                    outd.at[pl.ds(c * N + r0, WCH)])
                    pltpu.sync_copy(accde.at[pl.ds(r0, WCH)],
                                    vals.at[pl.ds(0, WCH)])
                    pltpu.sync_copy(vals.at[pl.ds(0, WCH)],
                                    outde.at[pl.ds(c * N + r0, WCH)])

    def wrap(*args):
        return pl.kernel(
            lambda *refs: body(refs),
            out_type=tuple(out_type) if with_deg else out_type[0],
            mesh=mesh,
            scratch_types=scratch,
        )(*args)

    return wrap


_sc_agg_deg = _make_sc_agg(True)
_sc_agg = _make_sc_agg(False)


BN = 1000
RB = N // BN


def _inv(v):
    return jnp.where(v > 0, 1.0 / jnp.where(v > 0, v, 1.0), 0.0)


def _scale_body(a0_ref, a1_ref, d0_ref, d1_ref, o_ref):
    deg = d0_ref[0, 0, 0, :] + d1_ref[0, 0, 0, :]
    o_ref[...] = (a0_ref[...] + a1_ref[...]) * _inv(deg)[:, None]


def _scale(P, degp):
    return pl.pallas_call(
        _scale_body,
        grid=(RB,),
        in_specs=[pl.BlockSpec((BN, D), lambda r: (r, 0)),
                  pl.BlockSpec((BN, D), lambda r: (RB + r, 0)),
                  pl.BlockSpec((1, 1, 1, BN), lambda r: (0, r, 0, 0)),
                  pl.BlockSpec((1, 1, 1, BN), lambda r: (1, r, 0, 0))],
        out_specs=pl.BlockSpec((BN, D), lambda r: (r, 0)),
        out_shape=jax.ShapeDtypeStruct((N, D), jnp.float32),
    )(P, P, degp, degp)


def _mmrelu_body(a0_ref, a1_ref, d0_ref, d1_ref, w_ref, b_ref, o_ref):
    d = d0_ref[0, 0, 0, :] + d1_ref[0, 0, 0, :]
    v = (a0_ref[...] + a1_ref[...]) * _inv(d)[:, None]
    o_ref[...] = jnp.maximum(
        jnp.dot(v, w_ref[...], preferred_element_type=jnp.float32)
        + b_ref[0, :][None, :], 0.0)


def _mmrelu(P, dp, W, b, DO):
    cb = DO // 128
    return pl.pallas_call(
        _mmrelu_body,
        grid=(cb, RB),
        in_specs=[pl.BlockSpec((BN, D), lambda c, r: (r, 0)),
                  pl.BlockSpec((BN, D), lambda c, r: (RB + r, 0)),
                  pl.BlockSpec((1, 1, 1, BN), lambda c, r: (0, r, 0, 0)),
                  pl.BlockSpec((1, 1, 1, BN), lambda c, r: (1, r, 0, 0)),
                  pl.BlockSpec((D, 128), lambda c, r: (0, c)),
                  pl.BlockSpec((1, 128), lambda c, r: (0, c))],
        out_specs=pl.BlockSpec((BN, 128), lambda c, r: (r, c)),
        out_shape=jax.ShapeDtypeStruct((N, DO), jnp.float32),
    )(P, P, dp, dp, W, b.reshape(1, DO))


def kernel(x, edge_index, edge_weight, batch, W1, b1, W2, b2):
    row = edge_index[0].astype(jnp.int32)
    col = edge_index[1].astype(jnp.int32)
    ew = edge_weight.astype(jnp.float32)

    P1, dpart, depart = _sc_agg_deg(x, row, col, ew)
    dp = dpart.reshape(2, RB, 1, BN)
    dep = depart.reshape(2, RB, 1, BN)
    T2 = _scale(P1, dep)
    P2 = _sc_agg(T2, col, row)
    h1 = _mmrelu(P2, dp, W1, b1, D)

    P3 = _sc_agg(h1, row, col)
    T4 = _scale(P3, dep)
    P4 = _sc_agg(T4, col, row)
    return _mmrelu(P4, dp, W2, b2, 2 * D)


# speedup vs baseline: 1.9952x; 1.2041x over previous
import functools

import jax
import jax.numpy as jnp
from jax import lax
from jax.experimental import pallas as pl
from jax.experimental.pallas import tpu as pltpu
from jax.experimental.pallas import tpu_sc as plsc

N = 10000
NNZ = 320000
D = 128

NC, NS, LANES = 2, 16, 16
CH = 128
EPC = NNZ // NC
NCHUNKS = EPC // CH
ITERS = (NCHUNKS + NS - 1) // NS
KD = D // LANES
WCH = 80
NWCH = N // WCH
WITER = (NWCH + NS - 1) // NS


def _zero_buf2d(buf, n):
    zval = jnp.zeros((LANES,), jnp.float32)

    def zrow(i, _):
        buf[i // KD, pl.ds((i % KD) * LANES, LANES)] = zval
        return 0

    lax.fori_loop(0, n * KD, zrow, 0)


def _make_sc_agg(with_deg):
    mesh = plsc.VectorSubcoreMesh(core_axis_name="c", subcore_axis_name="s")

    out_type = [jax.ShapeDtypeStruct((2 * N, D), jnp.float32)]
    scratch = [
        pltpu.VMEM((CH, D), jnp.float32),
        pltpu.VMEM((CH, D), jnp.float32),
        pltpu.VMEM((CH,), jnp.int32),
        pltpu.VMEM((CH,), jnp.int32),
        pltpu.VMEM((CH,), jnp.int32),
        pltpu.VMEM((CH,), jnp.int32),
        pltpu.VMEM_SHARED((N, D), jnp.float32),
        pltpu.SemaphoreType.DMA,
        pltpu.SemaphoreType.DMA,
        pltpu.SemaphoreType.DMA,
    ]
    if with_deg:
        out_type += [jax.ShapeDtypeStruct((2 * N,), jnp.float32),
                     jax.ShapeDtypeStruct((2 * N,), jnp.float32)]
        scratch += [
            pltpu.VMEM((CH,), jnp.float32),
            pltpu.VMEM((CH,), jnp.float32),
            pltpu.VMEM_SHARED((N,), jnp.float32),
            pltpu.VMEM_SHARED((N,), jnp.float32),
            pltpu.SemaphoreType.DMA,
        ]

    def body(refs):
        if with_deg:
            (table, gidx, sidx, ew, out, outd, outde,
             rows, rows1, gbuf, gb1, sbuf, sb1, acc, sem, semg1, sems,
             vals, ones, accd, accde, sem2) = refs
        else:
            (table, gidx, sidx, out,
             rows, rows1, gbuf, gb1, sbuf, sb1, acc, sem, semg1,
             sems) = refs
        c = lax.axis_index("c")
        s = lax.axis_index("s")

        _zero_buf2d(rows, CH)
        if with_deg:
            zv = jnp.zeros((LANES,), jnp.float32)
            ov = jnp.ones((LANES,), jnp.float32)
            for k in range(CH // LANES):
                vals[pl.ds(k * LANES, LANES)] = zv
                ones[pl.ds(k * LANES, LANES)] = ov
        for t in range(WITER):
            wid = t * NS + s

            @pl.when(wid < NWCH)
            def _():
                pltpu.sync_copy(rows.at[pl.ds(0, WCH)],
                                acc.at[pl.ds(wid * WCH, WCH)])
                if with_deg:
                    pltpu.sync_copy(vals.at[pl.ds(0, WCH)],
                                    accd.at[pl.ds(wid * WCH, WCH)])
                    pltpu.sync_copy(vals.at[pl.ds(0, WCH)],
                                    accde.at[pl.ds(wid * WCH, WCH)])

        plsc.subcore_barrier()

        if with_deg:
            def chunk(i, _):
                cid = i * NS + s

                @pl.when(cid < NCHUNKS)
                def _():
                    base = c * EPC + cid * CH
                    pltpu.sync_copy(gidx.at[pl.ds(base, CH)], gbuf)
                    g = pltpu.async_copy(table.at[gbuf], rows, sem)
                    pltpu.sync_copy(sidx.at[pl.ds(base, CH)], sbuf)
                    pltpu.async_copy(ew.at[sbuf], vals, sem2).wait()
                    pltpu.sync_copy(vals, accd.at[gbuf], add=True)
                    pltpu.sync_copy(ones, accde.at[sbuf], add=True)
                    g.wait()
                    pltpu.sync_copy(rows, acc.at[sbuf], add=True)

                return 0

            lax.fori_loop(0, ITERS, chunk, 0)
        else:
            def duo(m, _):
                c0 = (2 * m) * NS + s
                c1 = (2 * m + 1) * NS + s

                @pl.when(c0 < NCHUNKS)
                def _():
                    base = c * EPC + c0 * CH
                    pltpu.sync_copy(gidx.at[pl.ds(base, CH)], gbuf)
                    pltpu.async_copy(table.at[gbuf], rows, sem)
                    pltpu.sync_copy(sidx.at[pl.ds(base, CH)], sbuf)

                @pl.when(c1 < NCHUNKS)
                def _():
                    base = c * EPC + c1 * CH
                    pltpu.sync_copy(gidx.at[pl.ds(base, CH)], gb1)
                    pltpu.async_copy(table.at[gb1], rows1, semg1)
                    pltpu.sync_copy(sidx.at[pl.ds(base, CH)], sb1)

                @pl.when(c0 < NCHUNKS)
                def _():
                    pltpu.make_async_copy(table.at[gbuf], rows, sem).wait()
                    pltpu.async_copy(rows, acc.at[sbuf], sems, add=True)

                @pl.when(c1 < NCHUNKS)
                def _():
                    pltpu.make_async_copy(table.at[gb1], rows1,
                                          semg1).wait()
                    pltpu.sync_copy(rows1, acc.at[sb1], add=True)

                @pl.when(c0 < NCHUNKS)
                def _():
                    pltpu.make_async_copy(rows, acc.at[sbuf], sems).wait()

                return 0

            lax.fori_loop(0, (ITERS + 1) // 2, duo, 0)
        plsc.subcore_barrier()

        for t in range(WITER):
            wid = t * NS + s

            @pl.when(wid < NWCH)
            def _():
                r0 = wid * WCH
                pltpu.sync_copy(acc.at[pl.ds(r0, WCH)], rows.at[pl.ds(0, WCH)])
                pltpu.sync_copy(rows.at[pl.ds(0, WCH)],
                                out.at[pl.ds(c * N + r0, WCH)])
                if with_deg:
                    pltpu.sync_copy(accd.at[pl.ds(r0, WCH)],
                                    vals.at[pl.ds(0, WCH)])
                    pltpu.sync_copy(vals.at[pl.ds(0, WCH)],
                                    outd.at[pl.ds(c * N + r0, WCH)])
                    pltpu.sync_copy(accde.at[pl.ds(r0, WCH)],
                                    vals.at[pl.ds(0, WCH)])
                    pltpu.sync_copy(vals.at[pl.ds(0, WCH)],
                                    outde.at[pl.ds(c * N + r0, WCH)])

    def wrap(*args):
        return pl.kernel(
            lambda *refs: body(refs),
            out_type=tuple(out_type) if with_deg else out_type[0],
            mesh=mesh,
            scratch_types=scratch,
        )(*args)

    return wrap


_sc_agg_deg = _make_sc_agg(True)
_sc_agg = _make_sc_agg(False)


BN = 1000
RB = N // BN


def _inv(v):
    return jnp.where(v > 0, 1.0 / jnp.where(v > 0, v, 1.0), 0.0)


def _scale_body(a0_ref, a1_ref, d0_ref, d1_ref, o_ref):
    deg = d0_ref[0, 0, 0, :] + d1_ref[0, 0, 0, :]
    o_ref[...] = (a0_ref[...] + a1_ref[...]) * _inv(deg)[:, None]


def _scale(P, degp):
    return pl.pallas_call(
        _scale_body,
        grid=(RB,),
        in_specs=[pl.BlockSpec((BN, D), lambda r: (r, 0)),
                  pl.BlockSpec((BN, D), lambda r: (RB + r, 0)),
                  pl.BlockSpec((1, 1, 1, BN), lambda r: (0, r, 0, 0)),
                  pl.BlockSpec((1, 1, 1, BN), lambda r: (1, r, 0, 0))],
        out_specs=pl.BlockSpec((BN, D), lambda r: (r, 0)),
        out_shape=jax.ShapeDtypeStruct((N, D), jnp.float32),
    )(P, P, degp, degp)


def _mmrelu_body(a0_ref, a1_ref, d0_ref, d1_ref, w_ref, b_ref, o_ref):
    d = d0_ref[0, 0, 0, :] + d1_ref[0, 0, 0, :]
    v = (a0_ref[...] + a1_ref[...]) * _inv(d)[:, None]
    o_ref[...] = jnp.maximum(
        jnp.dot(v, w_ref[...], preferred_element_type=jnp.float32)
        + b_ref[0, :][None, :], 0.0)


def _mmrelu(P, dp, W, b, DO):
    cb = DO // 128
    return pl.pallas_call(
        _mmrelu_body,
        grid=(cb, RB),
        in_specs=[pl.BlockSpec((BN, D), lambda c, r: (r, 0)),
                  pl.BlockSpec((BN, D), lambda c, r: (RB + r, 0)),
                  pl.BlockSpec((1, 1, 1, BN), lambda c, r: (0, r, 0, 0)),
                  pl.BlockSpec((1, 1, 1, BN), lambda c, r: (1, r, 0, 0)),
                  pl.BlockSpec((D, 128), lambda c, r: (0, c)),
                  pl.BlockSpec((1, 128), lambda c, r: (0, c))],
        out_specs=pl.BlockSpec((BN, 128), lambda c, r: (r, c)),
        out_shape=jax.ShapeDtypeStruct((N, DO), jnp.float32),
    )(P, P, dp, dp, W, b.reshape(1, DO))


def kernel(x, edge_index, edge_weight, batch, W1, b1, W2, b2):
    row = edge_index[0].astype(jnp.int32)
    col = edge_index[1].astype(jnp.int32)
    ew = edge_weight.astype(jnp.float32)

    P1, dpart, depart = _sc_agg_deg(x, row, col, ew)
    dp = dpart.reshape(2, RB, 1, BN)
    dep = depart.reshape(2, RB, 1, BN)
    T2 = _scale(P1, dep)
    P2 = _sc_agg(T2, col, row)
    h1 = _mmrelu(P2, dp, W1, b1, D)

    P3 = _sc_agg(h1, row, col)
    T4 = _scale(P3, dep)
    P4 = _sc_agg(T4, col, row)
    return _mmrelu(P4, dp, W2, b2, 2 * D)


# speedup vs baseline: 2.1350x; 1.0701x over previous
import functools

import jax
import jax.numpy as jnp
from jax import lax
from jax.experimental import pallas as pl
from jax.experimental.pallas import tpu as pltpu
from jax.experimental.pallas import tpu_sc as plsc

N = 10000
NNZ = 320000
D = 128

NC, NS, LANES = 2, 16, 16
CH = 128
EPC = NNZ // NC
NCHUNKS = EPC // CH
ITERS = (NCHUNKS + NS - 1) // NS
KD = D // LANES
WCH = 80
NWCH = N // WCH
WITER = (NWCH + NS - 1) // NS


def _zero_buf2d(buf, n):
    zval = jnp.zeros((LANES,), jnp.float32)

    def zrow(i, _):
        buf[i // KD, pl.ds((i % KD) * LANES, LANES)] = zval
        return 0

    lax.fori_loop(0, n * KD, zrow, 0)


def _make_sc_agg(with_deg):
    mesh = plsc.VectorSubcoreMesh(core_axis_name="c", subcore_axis_name="s")

    out_type = [jax.ShapeDtypeStruct((2 * N, D), jnp.float32)]
    scratch = [
        pltpu.VMEM((CH, D), jnp.float32),
        pltpu.VMEM((CH, D), jnp.float32),
        pltpu.VMEM((CH,), jnp.int32),
        pltpu.VMEM((CH,), jnp.int32),
        pltpu.VMEM((CH,), jnp.int32),
        pltpu.VMEM((CH,), jnp.int32),
        pltpu.VMEM_SHARED((N, D), jnp.float32),
        pltpu.SemaphoreType.DMA,
        pltpu.SemaphoreType.DMA,
        pltpu.SemaphoreType.DMA,
    ]
    if with_deg:
        out_type += [jax.ShapeDtypeStruct((2 * N,), jnp.float32),
                     jax.ShapeDtypeStruct((2 * N,), jnp.float32)]
        scratch += [
            pltpu.VMEM((CH,), jnp.float32),
            pltpu.VMEM((CH,), jnp.float32),
            pltpu.VMEM_SHARED((N,), jnp.float32),
            pltpu.VMEM_SHARED((N,), jnp.float32),
            pltpu.SemaphoreType.DMA,
        ]

    def body(refs):
        if with_deg:
            (table, gidx, sidx, ew, out, outd, outde,
             rows, rows1, gbuf, gb1, sbuf, sb1, acc, sem, semg1, sems,
             vals, ones, accd, accde, sem2) = refs
        else:
            (table, gidx, sidx, out,
             rows, rows1, gbuf, gb1, sbuf, sb1, acc, sem, semg1,
             sems) = refs
        c = lax.axis_index("c")
        s = lax.axis_index("s")

        _zero_buf2d(rows, CH)
        if with_deg:
            zv = jnp.zeros((LANES,), jnp.float32)
            ov = jnp.ones((LANES,), jnp.float32)
            for k in range(CH // LANES):
                vals[pl.ds(k * LANES, LANES)] = zv
                ones[pl.ds(k * LANES, LANES)] = ov
        for t in range(WITER):
            wid = t * NS + s

            @pl.when(wid < NWCH)
            def _():
                pltpu.sync_copy(rows.at[pl.ds(0, WCH)],
                                acc.at[pl.ds(wid * WCH, WCH)])
                if with_deg:
                    pltpu.sync_copy(vals.at[pl.ds(0, WCH)],
                                    accd.at[pl.ds(wid * WCH, WCH)])
                    pltpu.sync_copy(vals.at[pl.ds(0, WCH)],
                                    accde.at[pl.ds(wid * WCH, WCH)])

        plsc.subcore_barrier()

        if with_deg:
            def duo_deg(m, _):
                c0 = (2 * m) * NS + s
                c1 = (2 * m + 1) * NS + s

                @pl.when(c0 < NCHUNKS)
                def _():
                    base = c * EPC + c0 * CH
                    pltpu.sync_copy(gidx.at[pl.ds(base, CH)], gbuf)
                    pltpu.async_copy(table.at[gbuf], rows, sem)
                    pltpu.sync_copy(sidx.at[pl.ds(base, CH)], sbuf)
                    pltpu.async_copy(ew.at[sbuf], vals, sem2)

                @pl.when(c1 < NCHUNKS)
                def _():
                    base = c * EPC + c1 * CH
                    pltpu.sync_copy(gidx.at[pl.ds(base, CH)], gb1)
                    pltpu.async_copy(table.at[gb1], rows1, semg1)
                    pltpu.sync_copy(sidx.at[pl.ds(base, CH)], sb1)

                @pl.when(c0 < NCHUNKS)
                def _():
                    pltpu.make_async_copy(ew.at[sbuf], vals, sem2).wait()
                    pltpu.sync_copy(vals, accd.at[gbuf], add=True)
                    pltpu.sync_copy(ones, accde.at[sbuf], add=True)
                    pltpu.make_async_copy(table.at[gbuf], rows, sem).wait()
                    pltpu.async_copy(rows, acc.at[sbuf], sems, add=True)

                @pl.when(c1 < NCHUNKS)
                def _():
                    pltpu.async_copy(ew.at[sb1], vals, sem2).wait()
                    pltpu.sync_copy(vals, accd.at[gb1], add=True)
                    pltpu.sync_copy(ones, accde.at[sb1], add=True)
                    pltpu.make_async_copy(table.at[gb1], rows1,
                                          semg1).wait()
                    pltpu.sync_copy(rows1, acc.at[sb1], add=True)

                @pl.when(c0 < NCHUNKS)
                def _():
                    pltpu.make_async_copy(rows, acc.at[sbuf], sems).wait()

                return 0

            lax.fori_loop(0, (ITERS + 1) // 2, duo_deg, 0)
        else:
            def duo(m, _):
                c0 = (2 * m) * NS + s
                c1 = (2 * m + 1) * NS + s

                @pl.when(c0 < NCHUNKS)
                def _():
                    base = c * EPC + c0 * CH
                    pltpu.sync_copy(gidx.at[pl.ds(base, CH)], gbuf)
                    pltpu.async_copy(table.at[gbuf], rows, sem)
                    pltpu.sync_copy(sidx.at[pl.ds(base, CH)], sbuf)

                @pl.when(c1 < NCHUNKS)
                def _():
                    base = c * EPC + c1 * CH
                    pltpu.sync_copy(gidx.at[pl.ds(base, CH)], gb1)
                    pltpu.async_copy(table.at[gb1], rows1, semg1)
                    pltpu.sync_copy(sidx.at[pl.ds(base, CH)], sb1)

                @pl.when(c0 < NCHUNKS)
                def _():
                    pltpu.make_async_copy(table.at[gbuf], rows, sem).wait()
                    pltpu.async_copy(rows, acc.at[sbuf], sems, add=True)

                @pl.when(c1 < NCHUNKS)
                def _():
                    pltpu.make_async_copy(table.at[gb1], rows1,
                                          semg1).wait()
                    pltpu.sync_copy(rows1, acc.at[sb1], add=True)

                @pl.when(c0 < NCHUNKS)
                def _():
                    pltpu.make_async_copy(rows, acc.at[sbuf], sems).wait()

                return 0

            lax.fori_loop(0, (ITERS + 1) // 2, duo, 0)
        plsc.subcore_barrier()

        for t in range(WITER):
            wid = t * NS + s

            @pl.when(wid < NWCH)
            def _():
                r0 = wid * WCH
                pltpu.sync_copy(acc.at[pl.ds(r0, WCH)], rows.at[pl.ds(0, WCH)])
                pltpu.sync_copy(rows.at[pl.ds(0, WCH)],
                                out.at[pl.ds(c * N + r0, WCH)])
                if with_deg:
                    pltpu.sync_copy(accd.at[pl.ds(r0, WCH)],
                                    vals.at[pl.ds(0, WCH)])
                    pltpu.sync_copy(vals.at[pl.ds(0, WCH)],
                                    outd.at[pl.ds(c * N + r0, WCH)])
                    pltpu.sync_copy(accde.at[pl.ds(r0, WCH)],
                                    vals.at[pl.ds(0, WCH)])
                    pltpu.sync_copy(vals.at[pl.ds(0, WCH)],
                                    outde.at[pl.ds(c * N + r0, WCH)])

    def wrap(*args):
        return pl.kernel(
            lambda *refs: body(refs),
            out_type=tuple(out_type) if with_deg else out_type[0],
            mesh=mesh,
            scratch_types=scratch,
        )(*args)

    return wrap


_sc_agg_deg = _make_sc_agg(True)
_sc_agg = _make_sc_agg(False)


BN = 1000
RB = N // BN


def _inv(v):
    return jnp.where(v > 0, 1.0 / jnp.where(v > 0, v, 1.0), 0.0)


def _scale_body(a0_ref, a1_ref, d0_ref, d1_ref, o_ref):
    deg = d0_ref[0, 0, 0, :] + d1_ref[0, 0, 0, :]
    o_ref[...] = (a0_ref[...] + a1_ref[...]) * _inv(deg)[:, None]


def _scale(P, degp):
    return pl.pallas_call(
        _scale_body,
        grid=(RB,),
        in_specs=[pl.BlockSpec((BN, D), lambda r: (r, 0)),
                  pl.BlockSpec((BN, D), lambda r: (RB + r, 0)),
                  pl.BlockSpec((1, 1, 1, BN), lambda r: (0, r, 0, 0)),
                  pl.BlockSpec((1, 1, 1, BN), lambda r: (1, r, 0, 0))],
        out_specs=pl.BlockSpec((BN, D), lambda r: (r, 0)),
        out_shape=jax.ShapeDtypeStruct((N, D), jnp.float32),
    )(P, P, degp, degp)


def _mmrelu_body(a0_ref, a1_ref, d0_ref, d1_ref, w_ref, b_ref, o_ref):
    d = d0_ref[0, 0, 0, :] + d1_ref[0, 0, 0, :]
    v = (a0_ref[...] + a1_ref[...]) * _inv(d)[:, None]
    o_ref[...] = jnp.maximum(
        jnp.dot(v, w_ref[...], preferred_element_type=jnp.float32)
        + b_ref[0, :][None, :], 0.0)


def _mmrelu(P, dp, W, b, DO):
    cb = DO // 128
    return pl.pallas_call(
        _mmrelu_body,
        grid=(cb, RB),
        in_specs=[pl.BlockSpec((BN, D), lambda c, r: (r, 0)),
                  pl.BlockSpec((BN, D), lambda c, r: (RB + r, 0)),
                  pl.BlockSpec((1, 1, 1, BN), lambda c, r: (0, r, 0, 0)),
                  pl.BlockSpec((1, 1, 1, BN), lambda c, r: (1, r, 0, 0)),
                  pl.BlockSpec((D, 128), lambda c, r: (0, c)),
                  pl.BlockSpec((1, 128), lambda c, r: (0, c))],
        out_specs=pl.BlockSpec((BN, 128), lambda c, r: (r, c)),
        out_shape=jax.ShapeDtypeStruct((N, DO), jnp.float32),
    )(P, P, dp, dp, W, b.reshape(1, DO))


def kernel(x, edge_index, edge_weight, batch, W1, b1, W2, b2):
    row = edge_index[0].astype(jnp.int32)
    col = edge_index[1].astype(jnp.int32)
    ew = edge_weight.astype(jnp.float32)

    P1, dpart, depart = _sc_agg_deg(x, row, col, ew)
    dp = dpart.reshape(2, RB, 1, BN)
    dep = depart.reshape(2, RB, 1, BN)
    T2 = _scale(P1, dep)
    P2 = _sc_agg(T2, col, row)
    h1 = _mmrelu(P2, dp, W1, b1, D)

    P3 = _sc_agg(h1, row, col)
    T4 = _scale(P3, dep)
    P4 = _sc_agg(T4, col, row)
    return _mmrelu(P4, dp, W2, b2, 2 * D)
